# baseline (speedup 1.0000x reference)
import functools

import jax
import jax.numpy as jnp
from jax import lax
from jax.experimental import pallas as pl
from jax.experimental.pallas import tpu as pltpu
from jax.experimental.pallas import tpu_sc as plsc

D = 128
B = 1024
I = 50
N = B * I
NC = 2
NS = 16
NW = NC * NS
PER_W = N // NW
C = 320
CHUNKS = PER_W // C
L = 16


def _sc_body(drug_hbm, dis_hbm, mp_hbm, out_hbm,
             mp_v, idx_v, out0, out1, sem0, sem1):
    cid = lax.axis_index("c")
    sid = lax.axis_index("s")
    wid = sid * NC + cid
    base0 = wid * PER_W

    pltpu.sync_copy(mp_hbm.at[pl.ds(base0 * 4, 4 * PER_W)], mp_v)
    lanes = lax.iota(jnp.int32, L)
    quarter = lanes >> 2
    perm0 = (4 * (lanes & 3)).astype(jnp.int32)

    def lane_take(v, perm):
        return lax.gather(
            v, perm[:, None],
            dimension_numbers=lax.GatherDimensionNumbers(
                offset_dims=(), collapsed_slice_dims=(0,),
                start_index_map=(0,)),
            slice_sizes=(1,),
            mode=lax.GatherScatterMode.PROMISE_IN_BOUNDS)

    def deint(t, carry):
        va = mp_v[pl.ds(t * 4 * L, L)]
        vb = mp_v[pl.ds(t * 4 * L + L, L)]
        vc = mp_v[pl.ds(t * 4 * L + 2 * L, L)]
        vd = mp_v[pl.ds(t * 4 * L + 3 * L, L)]
        for o in range(4):
            perm = perm0 + o
            g = jnp.where(
                quarter == 0, lane_take(va, perm),
                jnp.where(quarter == 1, lane_take(vb, perm),
                          jnp.where(quarter == 2, lane_take(vc, perm),
                                    lane_take(vd, perm))))
            idx_v[pl.ds(o * PER_W + t * L, L)] = g
        return carry

    lax.fori_loop(0, PER_W // L, deint, 0, unroll=4)

    zeros = jnp.zeros((L,), jnp.float32)

    def zero(out_v):
        def zbody(r, carry):
            for l in range(D // L):
                out_v[r, pl.ds(l * L, L)] = zeros
            return carry
        lax.fori_loop(0, C, zbody, 0, unroll=2)

    def start(chunk, out_v, sem):
        off = chunk * C
        for o, tbl in ((0, drug_hbm), (1, drug_hbm),
                       (2, dis_hbm), (3, dis_hbm)):
            pltpu.async_copy(
                tbl.at[idx_v.at[pl.ds(o * PER_W + off, C)]],
                out_v, sem, add=True)

    def wait(out_v, sem):
        for o, tbl in ((0, drug_hbm), (1, drug_hbm),
                       (2, dis_hbm), (3, dis_hbm)):
            pltpu.make_async_copy(
                tbl.at[idx_v.at[pl.ds(o * PER_W, C)]],
                out_v, sem).wait()

    def finish(chunk, out_v, sem):
        wait(out_v, sem)
        pltpu.sync_copy(out_v, out_hbm.at[pl.ds(base0 + chunk * C, C)])
        zero(out_v)

    zero(out0)
    zero(out1)
    start(0, out0, sem0)

    def pair_body(j, carry):
        c0 = 2 * j
        start(c0 + 1, out1, sem1)
        finish(c0, out0, sem0)
        start(c0 + 2, out0, sem0)
        finish(c0 + 1, out1, sem1)
        return carry

    lax.fori_loop(0, (CHUNKS - 1) // 2, pair_body, 0)
    wait(out0, sem0)
    pltpu.sync_copy(out0, out_hbm.at[pl.ds(base0 + (CHUNKS - 1) * C, C)])


@functools.partial(jax.jit, static_argnames=())
def _run(feature_drug, feature_disease, mp_flat):
    mesh = plsc.VectorSubcoreMesh(core_axis_name="c", subcore_axis_name="s")
    f = functools.partial(
        pl.kernel,
        mesh=mesh,
        out_type=jax.ShapeDtypeStruct((N, D), jnp.float32),
        scratch_types=[
            pltpu.VMEM((4 * PER_W,), jnp.int32),
            pltpu.VMEM((4 * PER_W,), jnp.int32),
            pltpu.VMEM((C, D), jnp.float32),
            pltpu.VMEM((C, D), jnp.float32),
            pltpu.SemaphoreType.DMA,
            pltpu.SemaphoreType.DMA,
        ],
    )(_sc_body)
    return f(feature_drug, feature_disease, mp_flat)


def kernel(feature_drug, feature_disease, mp_ins):
    mp_flat = mp_ins.astype(jnp.int32).reshape(N * 4)
    out = _run(feature_drug, feature_disease, mp_flat)
    return out.reshape(B, I, D)


# speedup vs baseline: 1.0223x; 1.0223x over previous
import functools

import jax
import jax.numpy as jnp
from jax import lax
from jax.experimental import pallas as pl
from jax.experimental.pallas import tpu as pltpu
from jax.experimental.pallas import tpu_sc as plsc

D = 128
B = 1024
I = 50
N = B * I
NC = 2
NS = 16
NW = NC * NS
PER_W = N // NW
C = 320
CHUNKS = PER_W // C
L = 16


def _sc_body(drug_hbm, dis_hbm, mp_hbm, out_hbm,
             mp_v, idx_v, out0, out1, sem0, sem1):
    cid = lax.axis_index("c")
    sid = lax.axis_index("s")
    wid = sid * NC + cid
    base0 = wid * PER_W

    pltpu.sync_copy(mp_hbm.at[pl.ds(base0 * 4, 4 * PER_W)], mp_v)
    lanes = lax.iota(jnp.int32, L)
    quarter = lanes >> 2
    perm0 = (4 * (lanes & 3)).astype(jnp.int32)

    def lane_take(v, perm):
        return lax.gather(
            v, perm[:, None],
            dimension_numbers=lax.GatherDimensionNumbers(
                offset_dims=(), collapsed_slice_dims=(0,),
                start_index_map=(0,)),
            slice_sizes=(1,),
            mode=lax.GatherScatterMode.PROMISE_IN_BOUNDS)

    def deint(t, carry):
        va = mp_v[pl.ds(t * 4 * L, L)]
        vb = mp_v[pl.ds(t * 4 * L + L, L)]
        vc = mp_v[pl.ds(t * 4 * L + 2 * L, L)]
        vd = mp_v[pl.ds(t * 4 * L + 3 * L, L)]
        for o in range(4):
            perm = perm0 + o
            g = jnp.where(
                quarter == 0, lane_take(va, perm),
                jnp.where(quarter == 1, lane_take(vb, perm),
                          jnp.where(quarter == 2, lane_take(vc, perm),
                                    lane_take(vd, perm))))
            idx_v[pl.ds(o * PER_W + t * L, L)] = g
        return carry

    zeros = jnp.zeros((L,), jnp.float32)

    def zero(out_v):
        def zbody(r, carry):
            for l in range(D // L):
                out_v[r, pl.ds(l * L, L)] = zeros
            return carry
        lax.fori_loop(0, C, zbody, 0, unroll=2)

    def start(chunk, out_v, sem):
        off = chunk * C
        for o, tbl in ((0, drug_hbm), (1, drug_hbm),
                       (2, dis_hbm), (3, dis_hbm)):
            pltpu.async_copy(
                tbl.at[idx_v.at[pl.ds(o * PER_W + off, C)]],
                out_v, sem, add=True)

    def wait(out_v, sem):
        for o, tbl in ((0, drug_hbm), (1, drug_hbm),
                       (2, dis_hbm), (3, dis_hbm)):
            pltpu.make_async_copy(
                tbl.at[idx_v.at[pl.ds(o * PER_W, C)]],
                out_v, sem).wait()

    def finish(chunk, out_v, sem):
        wait(out_v, sem)
        pltpu.sync_copy(out_v, out_hbm.at[pl.ds(base0 + chunk * C, C)])
        zero(out_v)

    lax.fori_loop(0, C // L, deint, 0, unroll=4)
    zero(out0)
    start(0, out0, sem0)
    lax.fori_loop(C // L, PER_W // L, deint, 0, unroll=4)
    zero(out1)

    def pair_body(j, carry):
        c0 = 2 * j
        start(c0 + 1, out1, sem1)
        finish(c0, out0, sem0)
        start(c0 + 2, out0, sem0)
        finish(c0 + 1, out1, sem1)
        return carry

    lax.fori_loop(0, (CHUNKS - 1) // 2, pair_body, 0)
    wait(out0, sem0)
    pltpu.sync_copy(out0, out_hbm.at[pl.ds(base0 + (CHUNKS - 1) * C, C)])


@functools.partial(jax.jit, static_argnames=())
def _run(feature_drug, feature_disease, mp_flat):
    mesh = plsc.VectorSubcoreMesh(core_axis_name="c", subcore_axis_name="s")
    f = functools.partial(
        pl.kernel,
        mesh=mesh,
        out_type=jax.ShapeDtypeStruct((N, D), jnp.float32),
        scratch_types=[
            pltpu.VMEM((4 * PER_W,), jnp.int32),
            pltpu.VMEM((4 * PER_W,), jnp.int32),
            pltpu.VMEM((C, D), jnp.float32),
            pltpu.VMEM((C, D), jnp.float32),
            pltpu.SemaphoreType.DMA,
            pltpu.SemaphoreType.DMA,
        ],
    )(_sc_body)
    return f(feature_drug, feature_disease, mp_flat)


def kernel(feature_drug, feature_disease, mp_ins):
    mp_flat = mp_ins.astype(jnp.int32).reshape(N * 4)
    out = _run(feature_drug, feature_disease, mp_flat)
    return out.reshape(B, I, D)
